# baseline (speedup 1.0000x reference)
import functools

import jax
import jax.numpy as jnp
from jax import lax
from jax.experimental import pallas as pl
from jax.experimental.pallas import tpu as pltpu
from jax.experimental.pallas import tpu_sc as plsc

_N = 10000
_NP = 10112
_RPT = _NP // 16
_E = 320000
_C = 128
_TCH = 80
_TEDGE = _C * _TCH
_EPAD = 32 * _TEDGE
_TCH2 = 160
_ALLE = 2 * _EPAD
_TPE = _ALLE // 32
_NV = _TPE // 16
_NB = 2
_KB = 2
_UN2 = _EPAD // 16 // (_KB * _C)
_UN1 = _EPAD // 32 // (_KB * _C)

_f32 = jnp.float32


def _mesh():
    return plsc.VectorSubcoreMesh(
        core_axis_name="c", subcore_axis_name="s", num_cores=2, num_subcores=16
    )


def _sc_degrees(src3, dst3):
    @functools.partial(
        pl.kernel,
        out_type=jax.ShapeDtypeStruct((4 * _NP,), _f32),
        mesh=_mesh(),
        compiler_params=pltpu.CompilerParams(use_tc_tiling_on_sc=False),
        scratch_types=[
            pltpu.VMEM((_TCH, _C), jnp.int32),
            pltpu.VMEM((_TCH, _C), jnp.int32),
            pltpu.VMEM((_C,), _f32),
            pltpu.VMEM((640,), _f32),
            pltpu.VMEM_SHARED((_NP,), _f32),
            pltpu.VMEM_SHARED((_NP,), _f32),
        ],
    )
    def k(src_h, dst_h, out_h, src_v, dst_v, ones_v, stage_v, acc_o, acc_i):
        c = lax.axis_index("c")
        s = lax.axis_index("s")
        wid = c * 16 + s
        r0 = s * _RPT

        def zfill(i, carry):
            stage_v[pl.ds(i * 16, 16)] = jnp.zeros((16,), _f32)
            return carry

        lax.fori_loop(0, 640 // 16, zfill, 0)
        pltpu.sync_copy(stage_v.at[pl.ds(0, _RPT)], acc_o.at[pl.ds(r0, _RPT)])
        pltpu.sync_copy(stage_v.at[pl.ds(0, _RPT)], acc_i.at[pl.ds(r0, _RPT)])
        pltpu.sync_copy(src_h.at[wid], src_v)
        pltpu.sync_copy(dst_h.at[wid], dst_v)
        for t in range(_C // 16):
            ones_v[pl.ds(t * 16, 16)] = jnp.ones((16,), _f32)
        plsc.subcore_barrier()

        def body(j, carry):
            pltpu.sync_copy(ones_v, acc_o.at[src_v.at[j]], add=True)
            pltpu.sync_copy(ones_v, acc_i.at[dst_v.at[j]], add=True)
            return carry

        lax.fori_loop(0, _TCH, body, 0)
        plsc.subcore_barrier()
        pltpu.sync_copy(acc_o.at[pl.ds(r0, _RPT)], stage_v.at[pl.ds(0, _RPT)])
        pltpu.sync_copy(stage_v.at[pl.ds(0, _RPT)],
                        out_h.at[pl.ds((2 * c) * _NP + r0, _RPT)])
        pltpu.sync_copy(acc_i.at[pl.ds(r0, _RPT)], stage_v.at[pl.ds(0, _RPT)])
        pltpu.sync_copy(stage_v.at[pl.ds(0, _RPT)],
                        out_h.at[pl.ds((2 * c + 1) * _NP + r0, _RPT)])

    return k(src3, dst3)


def _sc_message_pass_split(tabA, tabB, src3, dst3, dh, kb, units):
    @functools.partial(
        pl.kernel,
        out_type=jax.ShapeDtypeStruct((2 * _NP, dh), _f32),
        mesh=_mesh(),
        compiler_params=pltpu.CompilerParams(use_tc_tiling_on_sc=False),
        scratch_types=[
            pltpu.VMEM((units, kb * _C), jnp.int32),
            pltpu.VMEM((units * kb, _C), jnp.int32),
            pltpu.VMEM((_NB, kb * _C, dh), _f32),
            pltpu.VMEM_SHARED((_NP, dh), _f32),
        ] + [pltpu.SemaphoreType.DMA] * _NB,
    )
    def k(tabA_h, tabB_h, src_h, dst_h, out_h,
          src_v, dst_v, rows_v, acc_s, *sems):
        c = lax.axis_index("c")
        s = lax.axis_index("s")
        r0 = s * _RPT
        slot = kb * _C
        parts = []
        off = 0
        while off < _RPT:
            ln = min(slot, _RPT - off)
            parts.append((off, ln))
            off += ln

        def zfill(i, carry):
            for t in range(dh // 16):
                rows_v[0, i, pl.ds(t * 16, 16)] = jnp.zeros((16,), _f32)
            return carry

        lax.fori_loop(0, slot, zfill, 0)
        for po, ln in parts:
            pltpu.sync_copy(rows_v.at[0, pl.ds(0, ln)],
                            acc_s.at[pl.ds(r0 + po, ln)])
        pltpu.sync_copy(src_h.at[s], src_v)
        pltpu.sync_copy(dst_h.at[s], dst_v)
        plsc.subcore_barrier()

        def edge_loop(tab_h):
            for p in range(_NB - 1):
                pltpu.async_copy(tab_h.at[src_v.at[p]], rows_v.at[p], sems[p])

            def body(jj, carry):
                for u in range(_NB):
                    j = jj * _NB + u
                    b2 = (u + _NB - 1) % _NB
                    j2 = j + _NB - 1

                    @pl.when(j2 < units)
                    def _():
                        pltpu.async_copy(tab_h.at[src_v.at[j2]],
                                         rows_v.at[b2], sems[b2])

                    pltpu.make_async_copy(tab_h.at[src_v.at[j]],
                                          rows_v.at[u], sems[u]).wait()
                    for q in range(kb):
                        pltpu.sync_copy(rows_v.at[u, pl.ds(q * _C, _C)],
                                        acc_s.at[dst_v.at[j * kb + q]],
                                        add=True)
                return carry

            lax.fori_loop(0, units // _NB, body, 0)

        @pl.when(c == 0)
        def _():
            edge_loop(tabA_h)

        @pl.when(c == 1)
        def _():
            edge_loop(tabB_h)

        plsc.subcore_barrier()
        for po, ln in parts:
            pltpu.sync_copy(acc_s.at[pl.ds(r0 + po, ln)],
                            rows_v.at[0, pl.ds(0, ln)])
            pltpu.sync_copy(rows_v.at[0, pl.ds(0, ln)],
                            out_h.at[pl.ds(c * _NP + r0 + po, ln)])

    return k(tabA, tabB, src3, dst3)


def _sc_message_pass_edges(tab, src3, dst3, d_feat, kb, units):
    @functools.partial(
        pl.kernel,
        out_type=jax.ShapeDtypeStruct((2 * _NP, d_feat), _f32),
        mesh=_mesh(),
        compiler_params=pltpu.CompilerParams(use_tc_tiling_on_sc=False),
        scratch_types=[
            pltpu.VMEM((units, kb * _C), jnp.int32),
            pltpu.VMEM((units * kb, _C), jnp.int32),
            pltpu.VMEM((_NB, kb * _C, d_feat), _f32),
            pltpu.VMEM_SHARED((_NP, d_feat), _f32),
        ] + [pltpu.SemaphoreType.DMA] * _NB,
    )
    def k(tab_h, src_h, dst_h, out_h, src_v, dst_v, rows_v, acc_s,
          *sems):
        c = lax.axis_index("c")
        s = lax.axis_index("s")
        wid = c * 16 + s
        r0 = s * _RPT
        slot = kb * _C
        parts = []
        off = 0
        while off < _RPT:
            ln = min(slot, _RPT - off)
            parts.append((off, ln))
            off += ln

        def zfill(i, carry):
            for t in range(d_feat // 16):
                rows_v[0, i, pl.ds(t * 16, 16)] = jnp.zeros((16,), _f32)
            return carry

        lax.fori_loop(0, slot, zfill, 0)
        for po, ln in parts:
            pltpu.sync_copy(rows_v.at[0, pl.ds(0, ln)],
                            acc_s.at[pl.ds(r0 + po, ln)])
        pltpu.sync_copy(src_h.at[wid], src_v)
        pltpu.sync_copy(dst_h.at[wid], dst_v)
        plsc.subcore_barrier()

        for p in range(_NB - 1):
            pltpu.async_copy(tab_h.at[src_v.at[p]], rows_v.at[p], sems[p])

        def body(jj, carry):
            for u in range(_NB):
                j = jj * _NB + u
                b2 = (u + _NB - 1) % _NB
                j2 = j + _NB - 1

                @pl.when(j2 < units)
                def _():
                    pltpu.async_copy(tab_h.at[src_v.at[j2]],
                                     rows_v.at[b2], sems[b2])

                pltpu.make_async_copy(tab_h.at[src_v.at[j]],
                                      rows_v.at[u], sems[u]).wait()
                for q in range(kb):
                    pltpu.sync_copy(rows_v.at[u, pl.ds(q * _C, _C)],
                                    acc_s.at[dst_v.at[j * kb + q]], add=True)
            return carry

        lax.fori_loop(0, units // _NB, body, 0)
        plsc.subcore_barrier()
        for po, ln in parts:
            pltpu.sync_copy(acc_s.at[pl.ds(r0 + po, ln)],
                            rows_v.at[0, pl.ds(0, ln)])
            pltpu.sync_copy(rows_v.at[0, pl.ds(0, ln)],
                            out_h.at[pl.ds(c * _NP + r0 + po, ln)])

    return k(tab, src3, dst3)


def _sc_edge_scores(ssrc, sdst, esrc, edst):
    @functools.partial(
        pl.kernel,
        out_type=jax.ShapeDtypeStruct((_ALLE,), _f32),
        mesh=_mesh(),
        compiler_params=pltpu.CompilerParams(needs_layout_passes=False),
        scratch_types=[
            pltpu.VMEM((_NP,), _f32),
            pltpu.VMEM((_NP,), _f32),
            pltpu.VMEM((_TPE,), jnp.int32),
            pltpu.VMEM((_TPE,), jnp.int32),
            pltpu.VMEM((_TPE,), _f32),
        ],
    )
    def k(ss_h, sd_h, es_h, ed_h, out_h, ss_v, sd_v, es_v, ed_v, out_v):
        c = lax.axis_index("c")
        s = lax.axis_index("s")
        wid = c * 16 + s
        base = wid * _TPE
        pltpu.sync_copy(ss_h, ss_v)
        pltpu.sync_copy(sd_h, sd_v)
        pltpu.sync_copy(es_h.at[pl.ds(base, _TPE)], es_v)
        pltpu.sync_copy(ed_h.at[pl.ds(base, _TPE)], ed_v)

        def body(i, carry):
            si = es_v[pl.ds(i * 16, 16)]
            di = ed_v[pl.ds(i * 16, 16)]
            out_v[pl.ds(i * 16, 16)] = (
                plsc.load_gather(ss_v, [si]) + plsc.load_gather(sd_v, [di])
            )
            return carry

        lax.fori_loop(0, _NV, body, 0)
        pltpu.sync_copy(out_v, out_h.at[pl.ds(base, _TPE)])

    return k(ssrc, sdst, esrc, edst)


def _tc_call(fn, out_shapes, *args):
    return pl.pallas_call(
        fn,
        out_shape=out_shapes,
        compiler_params=pltpu.CompilerParams(vmem_limit_bytes=100 * 1024 * 1024),
    )(*args)


def _tc_pre(degp, nfp, Wx, bx):
    def f(d_ref, nf_ref, wx_ref, bx_ref, h0sA_ref, h0sB_ref, x_ref,
          nrm_ref):
        d = d_ref[...]
        deg_o = d[:, 0:1] + d[:, 2:3]
        deg_i = d[:, 1:2] + d[:, 3:4]
        rows = lax.broadcasted_iota(jnp.int32, (_NP, 1), 0)
        valid = rows < _N
        no = jnp.where(valid, lax.rsqrt(jnp.maximum(deg_o, 1.0)), 0.0)
        ni = jnp.where(valid, lax.rsqrt(jnp.maximum(deg_i, 1.0)), 0.0)
        nf = nf_ref[...]
        h0s = nf * no
        h0sA_ref[...] = h0s[:, 0:64]
        h0sB_ref[...] = h0s[:, 64:128]
        x_ref[...] = jax.nn.relu(
            jnp.dot(nf, wx_ref[...], preferred_element_type=_f32) + bx_ref[...]
        )
        nrm_ref[...] = jnp.concatenate([no, ni], axis=1)

    return _tc_call(
        f,
        (
            jax.ShapeDtypeStruct((_NP, 64), _f32),
            jax.ShapeDtypeStruct((_NP, 64), _f32),
            jax.ShapeDtypeStruct((_NP, 64), _f32),
            jax.ShapeDtypeStruct((_NP, 2), _f32),
        ),
        degp, nfp, Wx, bx,
    )


def _tc_layer(aggs, nrm, W, b):
    def f(a_ref, nrm_ref, w_ref, b_ref, outA_ref, outB_ref):
        no = nrm_ref[:, 0:1]
        ni = nrm_ref[:, 1:2]
        p = jnp.concatenate([a_ref[0:_NP, :], a_ref[_NP:2 * _NP, :]], axis=1)
        h = jax.nn.relu(
            jnp.dot(p * ni, w_ref[...], preferred_element_type=_f32)
            + b_ref[...]
        )
        h = h * no
        outA_ref[...] = h[:, 0:64]
        outB_ref[...] = h[:, 64:128]

    return _tc_call(
        f,
        (
            jax.ShapeDtypeStruct((_NP, 64), _f32),
            jax.ShapeDtypeStruct((_NP, 64), _f32),
        ),
        aggs, nrm, W, b,
    )


def _tc_layer2(aggs, nrm, W1, b1, W2):
    def f(a_ref, nrm_ref, w1_ref, b1_ref, w2_ref, outA_ref, outB_ref):
        no = nrm_ref[:, 0:1]
        ni = nrm_ref[:, 1:2]
        p = jnp.concatenate([a_ref[0:_NP, :], a_ref[_NP:2 * _NP, :]], axis=1)
        h = jax.nn.relu(
            jnp.dot(p * ni, w1_ref[...], preferred_element_type=_f32)
            + b1_ref[...]
        )
        t2 = jnp.dot(h, w2_ref[...], preferred_element_type=_f32) * no
        outA_ref[...] = t2[:, 0:16]
        outB_ref[...] = t2[:, 16:32]

    return _tc_call(
        f,
        (
            jax.ShapeDtypeStruct((_NP, 16), _f32),
            jax.ShapeDtypeStruct((_NP, 16), _f32),
        ),
        aggs, nrm, W1, b1, W2,
    )


def _tc_aspect(aggs, nrm, x, b2, W3, We, be):
    def f(a_ref, nrm_ref, x_ref, b2_ref, w3_ref, we_ref, be_ref,
          asp_ref, t3_ref, sb_ref):
        no = nrm_ref[:, 0:1]
        ni = nrm_ref[:, 1:2]
        agg = jnp.concatenate([a_ref[0:_NP, :], a_ref[_NP:2 * _NP, :]], axis=1)
        ae = agg * ni + b2_ref[...]
        m = jnp.max(ae, axis=1, keepdims=True)
        lse = m + jnp.log(jnp.sum(jnp.exp(ae - m), axis=1, keepdims=True))
        asp = ae - lse
        asp_ref[...] = asp
        t3_ref[...] = jnp.dot(ae, w3_ref[...], preferred_element_type=_f32) * no
        we = we_ref[...]
        x = x_ref[...]
        ss = (
            jnp.dot(x, we[0:64, :], preferred_element_type=_f32)
            + jnp.dot(asp, we[128:160, :], preferred_element_type=_f32)
            + be_ref[...]
        )
        sd = (
            jnp.dot(x, we[64:128, :], preferred_element_type=_f32)
            + jnp.dot(asp, we[160:192, :], preferred_element_type=_f32)
        )
        sb_ref[...] = jnp.concatenate([ss, sd], axis=1)

    return _tc_call(
        f,
        (
            jax.ShapeDtypeStruct((_NP, 32), _f32),
            jax.ShapeDtypeStruct((_NP, 16), _f32),
            jax.ShapeDtypeStruct((_NP, 2), _f32),
        ),
        aggs, nrm, x, b2, W3, We, be,
    )


def _tc_logits(aggs, nrm, b3):
    def f(a_ref, nrm_ref, b3_ref, out_ref):
        agg = a_ref[0:_NP, :] + a_ref[_NP:2 * _NP, :]
        out_ref[...] = agg * nrm_ref[:, 1:2] + b3_ref[...]

    return _tc_call(f, jax.ShapeDtypeStruct((_NP, 16), _f32), aggs, nrm, b3)


def kernel(node_features, edge_index, edge_index_neg,
           W0, b0, W1, b1, W2, b2, W3, b3, Wx, bx, We, be):
    i32 = jnp.int32
    pad = jnp.full((_EPAD - _E,), _N, i32)
    src = jnp.concatenate([edge_index[0], pad])
    dst = jnp.concatenate([edge_index[1], pad])
    src3 = src.reshape(32, _TCH, _C)
    dst3 = dst.reshape(32, _TCH, _C)
    src3k = src.reshape(32, 20, 512)
    dst3k = dst.reshape(32, 80, _C)
    srcL2 = src.reshape(16, 80, 256)
    dstL = dst.reshape(16, 160, _C)
    srcL4 = src.reshape(16, 40, 512)
    nsrc = jnp.concatenate([edge_index_neg[0], pad])
    ndst = jnp.concatenate([edge_index_neg[1], pad])
    esrc = jnp.concatenate([src, nsrc])
    edst = jnp.concatenate([dst, ndst])

    nfp = jnp.zeros((_NP, 128), _f32).at[:_N].set(node_features)

    degp = _sc_degrees(src3, dst3).reshape(4, _NP).T
    h0sA, h0sB, x, nrm = _tc_pre(degp, nfp, Wx, bx)

    agg0 = _sc_message_pass_split(h0sA, h0sB, srcL2, dstL, 64, 2, 80)
    h1sA, h1sB = _tc_layer(agg0, nrm, W0, b0)
    agg1 = _sc_message_pass_split(h1sA, h1sB, srcL2, dstL, 64, 2, 80)
    t2A, t2B = _tc_layer2(agg1, nrm, W1, b1, W2)
    agg2 = _sc_message_pass_split(t2A, t2B, srcL4, dstL, 16, 4, 40)
    asp, t3, sboth = _tc_aspect(agg2, nrm, x, b2, W3, We, be)
    agg3 = _sc_message_pass_edges(t3, src3k, dst3k, 16, 4, 20)
    logits = _tc_logits(agg3, nrm, b3)

    preds = _sc_edge_scores(sboth[:, 0].reshape(_NP), sboth[:, 1].reshape(_NP),
                            esrc, edst)
    e_pred_pos = preds[:_E, None]
    e_pred_neg = preds[_EPAD:_EPAD + _E, None]
    return (e_pred_pos, e_pred_neg, asp[:_N], logits[:_N])


# speedup vs baseline: 1.0048x; 1.0048x over previous
import functools

import jax
import jax.numpy as jnp
from jax import lax
from jax.experimental import pallas as pl
from jax.experimental.pallas import tpu as pltpu
from jax.experimental.pallas import tpu_sc as plsc

_N = 10000
_NP = 10112
_RPT = _NP // 16
_E = 320000
_C = 128
_TCH = 80
_TEDGE = _C * _TCH
_EPAD = 32 * _TEDGE
_TCH2 = 160
_ALLE = 2 * _EPAD
_TPE = _ALLE // 32
_NV = _TPE // 16
_NB = 2
_KB = 2
_UN2 = _EPAD // 16 // (_KB * _C)
_UN1 = _EPAD // 32 // (_KB * _C)

_f32 = jnp.float32


def _mesh():
    return plsc.VectorSubcoreMesh(
        core_axis_name="c", subcore_axis_name="s", num_cores=2, num_subcores=16
    )


def _sc_degrees(src3, dst3):
    @functools.partial(
        pl.kernel,
        out_type=jax.ShapeDtypeStruct((4 * _NP,), _f32),
        mesh=_mesh(),
        compiler_params=pltpu.CompilerParams(use_tc_tiling_on_sc=False),
        scratch_types=[
            pltpu.VMEM((_TCH, _C), jnp.int32),
            pltpu.VMEM((_TCH, _C), jnp.int32),
            pltpu.VMEM((_C,), _f32),
            pltpu.VMEM((640,), _f32),
            pltpu.VMEM_SHARED((_NP,), _f32),
            pltpu.VMEM_SHARED((_NP,), _f32),
        ],
    )
    def k(src_h, dst_h, out_h, src_v, dst_v, ones_v, stage_v, acc_o, acc_i):
        c = lax.axis_index("c")
        s = lax.axis_index("s")
        wid = c * 16 + s
        r0 = s * _RPT

        def zfill(i, carry):
            stage_v[pl.ds(i * 16, 16)] = jnp.zeros((16,), _f32)
            return carry

        lax.fori_loop(0, 640 // 16, zfill, 0)
        pltpu.sync_copy(stage_v.at[pl.ds(0, _RPT)], acc_o.at[pl.ds(r0, _RPT)])
        pltpu.sync_copy(stage_v.at[pl.ds(0, _RPT)], acc_i.at[pl.ds(r0, _RPT)])
        pltpu.sync_copy(src_h.at[wid], src_v)
        pltpu.sync_copy(dst_h.at[wid], dst_v)
        for t in range(_C // 16):
            ones_v[pl.ds(t * 16, 16)] = jnp.ones((16,), _f32)
        plsc.subcore_barrier()

        def body(j, carry):
            pltpu.sync_copy(ones_v, acc_o.at[src_v.at[j]], add=True)
            pltpu.sync_copy(ones_v, acc_i.at[dst_v.at[j]], add=True)
            return carry

        lax.fori_loop(0, _TCH, body, 0)
        plsc.subcore_barrier()
        pltpu.sync_copy(acc_o.at[pl.ds(r0, _RPT)], stage_v.at[pl.ds(0, _RPT)])
        pltpu.sync_copy(stage_v.at[pl.ds(0, _RPT)],
                        out_h.at[pl.ds((2 * c) * _NP + r0, _RPT)])
        pltpu.sync_copy(acc_i.at[pl.ds(r0, _RPT)], stage_v.at[pl.ds(0, _RPT)])
        pltpu.sync_copy(stage_v.at[pl.ds(0, _RPT)],
                        out_h.at[pl.ds((2 * c + 1) * _NP + r0, _RPT)])

    return k(src3, dst3)


def _sc_message_pass_split(tabA, tabB, src3, dst3, dh, kb, units):
    @functools.partial(
        pl.kernel,
        out_type=jax.ShapeDtypeStruct((2 * _NP, dh), _f32),
        mesh=_mesh(),
        compiler_params=pltpu.CompilerParams(use_tc_tiling_on_sc=False),
        scratch_types=[
            pltpu.VMEM((units, kb * _C), jnp.int32),
            pltpu.VMEM((units * kb, _C), jnp.int32),
            pltpu.VMEM((_NB, kb * _C, dh), _f32),
            pltpu.VMEM_SHARED((_NP, dh), _f32),
        ] + [pltpu.SemaphoreType.DMA] * (_NB + kb - 1),
    )
    def k(tabA_h, tabB_h, src_h, dst_h, out_h,
          src_v, dst_v, rows_v, acc_s, *sems):
        c = lax.axis_index("c")
        s = lax.axis_index("s")
        r0 = s * _RPT
        slot = kb * _C
        parts = []
        off = 0
        while off < _RPT:
            ln = min(slot, _RPT - off)
            parts.append((off, ln))
            off += ln

        def zfill(i, carry):
            for t in range(dh // 16):
                rows_v[0, i, pl.ds(t * 16, 16)] = jnp.zeros((16,), _f32)
            return carry

        lax.fori_loop(0, slot, zfill, 0)
        for po, ln in parts:
            pltpu.sync_copy(rows_v.at[0, pl.ds(0, ln)],
                            acc_s.at[pl.ds(r0 + po, ln)])
        pltpu.sync_copy(src_h.at[s], src_v)
        pltpu.sync_copy(dst_h.at[s], dst_v)
        plsc.subcore_barrier()

        def edge_loop(tab_h):
            for p in range(_NB - 1):
                pltpu.async_copy(tab_h.at[src_v.at[p]], rows_v.at[p], sems[p])

            def body(jj, carry):
                for u in range(_NB):
                    j = jj * _NB + u
                    b2 = (u + _NB - 1) % _NB
                    j2 = j + _NB - 1

                    @pl.when(j2 < units)
                    def _():
                        pltpu.async_copy(tab_h.at[src_v.at[j2]],
                                         rows_v.at[b2], sems[b2])

                    pltpu.make_async_copy(tab_h.at[src_v.at[j]],
                                          rows_v.at[u], sems[u]).wait()
                    cps = [
                        pltpu.async_copy(rows_v.at[u, pl.ds(q * _C, _C)],
                                         acc_s.at[dst_v.at[j * kb + q]],
                                         sems[_NB + q], add=True)
                        for q in range(kb - 1)
                    ]
                    pltpu.sync_copy(rows_v.at[u, pl.ds((kb - 1) * _C, _C)],
                                    acc_s.at[dst_v.at[j * kb + kb - 1]],
                                    add=True)
                    for cp in cps:
                        cp.wait()
                return carry

            lax.fori_loop(0, units // _NB, body, 0)

        @pl.when(c == 0)
        def _():
            edge_loop(tabA_h)

        @pl.when(c == 1)
        def _():
            edge_loop(tabB_h)

        plsc.subcore_barrier()
        for po, ln in parts:
            pltpu.sync_copy(acc_s.at[pl.ds(r0 + po, ln)],
                            rows_v.at[0, pl.ds(0, ln)])
            pltpu.sync_copy(rows_v.at[0, pl.ds(0, ln)],
                            out_h.at[pl.ds(c * _NP + r0 + po, ln)])

    return k(tabA, tabB, src3, dst3)


def _sc_message_pass_edges(tab, src3, dst3, d_feat, kb, units):
    @functools.partial(
        pl.kernel,
        out_type=jax.ShapeDtypeStruct((2 * _NP, d_feat), _f32),
        mesh=_mesh(),
        compiler_params=pltpu.CompilerParams(use_tc_tiling_on_sc=False),
        scratch_types=[
            pltpu.VMEM((units, kb * _C), jnp.int32),
            pltpu.VMEM((units * kb, _C), jnp.int32),
            pltpu.VMEM((_NB, kb * _C, d_feat), _f32),
            pltpu.VMEM_SHARED((_NP, d_feat), _f32),
        ] + [pltpu.SemaphoreType.DMA] * (_NB + kb - 1),
    )
    def k(tab_h, src_h, dst_h, out_h, src_v, dst_v, rows_v, acc_s,
          *sems):
        c = lax.axis_index("c")
        s = lax.axis_index("s")
        wid = c * 16 + s
        r0 = s * _RPT
        slot = kb * _C
        parts = []
        off = 0
        while off < _RPT:
            ln = min(slot, _RPT - off)
            parts.append((off, ln))
            off += ln

        def zfill(i, carry):
            for t in range(d_feat // 16):
                rows_v[0, i, pl.ds(t * 16, 16)] = jnp.zeros((16,), _f32)
            return carry

        lax.fori_loop(0, slot, zfill, 0)
        for po, ln in parts:
            pltpu.sync_copy(rows_v.at[0, pl.ds(0, ln)],
                            acc_s.at[pl.ds(r0 + po, ln)])
        pltpu.sync_copy(src_h.at[wid], src_v)
        pltpu.sync_copy(dst_h.at[wid], dst_v)
        plsc.subcore_barrier()

        for p in range(_NB - 1):
            pltpu.async_copy(tab_h.at[src_v.at[p]], rows_v.at[p], sems[p])

        def body(jj, carry):
            for u in range(_NB):
                j = jj * _NB + u
                b2 = (u + _NB - 1) % _NB
                j2 = j + _NB - 1

                @pl.when(j2 < units)
                def _():
                    pltpu.async_copy(tab_h.at[src_v.at[j2]],
                                     rows_v.at[b2], sems[b2])

                pltpu.make_async_copy(tab_h.at[src_v.at[j]],
                                      rows_v.at[u], sems[u]).wait()
                cps = [
                    pltpu.async_copy(rows_v.at[u, pl.ds(q * _C, _C)],
                                     acc_s.at[dst_v.at[j * kb + q]],
                                     sems[_NB + q], add=True)
                    for q in range(kb - 1)
                ]
                pltpu.sync_copy(rows_v.at[u, pl.ds((kb - 1) * _C, _C)],
                                acc_s.at[dst_v.at[j * kb + kb - 1]], add=True)
                for cp in cps:
                    cp.wait()
            return carry

        lax.fori_loop(0, units // _NB, body, 0)
        plsc.subcore_barrier()
        for po, ln in parts:
            pltpu.sync_copy(acc_s.at[pl.ds(r0 + po, ln)],
                            rows_v.at[0, pl.ds(0, ln)])
            pltpu.sync_copy(rows_v.at[0, pl.ds(0, ln)],
                            out_h.at[pl.ds(c * _NP + r0 + po, ln)])

    return k(tab, src3, dst3)


def _sc_edge_scores(ssrc, sdst, esrc, edst):
    @functools.partial(
        pl.kernel,
        out_type=jax.ShapeDtypeStruct((_ALLE,), _f32),
        mesh=_mesh(),
        compiler_params=pltpu.CompilerParams(needs_layout_passes=False),
        scratch_types=[
            pltpu.VMEM((_NP,), _f32),
            pltpu.VMEM((_NP,), _f32),
            pltpu.VMEM((_TPE,), jnp.int32),
            pltpu.VMEM((_TPE,), jnp.int32),
            pltpu.VMEM((_TPE,), _f32),
        ],
    )
    def k(ss_h, sd_h, es_h, ed_h, out_h, ss_v, sd_v, es_v, ed_v, out_v):
        c = lax.axis_index("c")
        s = lax.axis_index("s")
        wid = c * 16 + s
        base = wid * _TPE
        pltpu.sync_copy(ss_h, ss_v)
        pltpu.sync_copy(sd_h, sd_v)
        pltpu.sync_copy(es_h.at[pl.ds(base, _TPE)], es_v)
        pltpu.sync_copy(ed_h.at[pl.ds(base, _TPE)], ed_v)

        def body(i, carry):
            si = es_v[pl.ds(i * 16, 16)]
            di = ed_v[pl.ds(i * 16, 16)]
            out_v[pl.ds(i * 16, 16)] = (
                plsc.load_gather(ss_v, [si]) + plsc.load_gather(sd_v, [di])
            )
            return carry

        lax.fori_loop(0, _NV, body, 0)
        pltpu.sync_copy(out_v, out_h.at[pl.ds(base, _TPE)])

    return k(ssrc, sdst, esrc, edst)


def _tc_call(fn, out_shapes, *args):
    return pl.pallas_call(
        fn,
        out_shape=out_shapes,
        compiler_params=pltpu.CompilerParams(vmem_limit_bytes=100 * 1024 * 1024),
    )(*args)


def _tc_pre(degp, nfp, Wx, bx):
    def f(d_ref, nf_ref, wx_ref, bx_ref, h0sA_ref, h0sB_ref, x_ref,
          nrm_ref):
        d = d_ref[...]
        deg_o = d[:, 0:1] + d[:, 2:3]
        deg_i = d[:, 1:2] + d[:, 3:4]
        rows = lax.broadcasted_iota(jnp.int32, (_NP, 1), 0)
        valid = rows < _N
        no = jnp.where(valid, lax.rsqrt(jnp.maximum(deg_o, 1.0)), 0.0)
        ni = jnp.where(valid, lax.rsqrt(jnp.maximum(deg_i, 1.0)), 0.0)
        nf = nf_ref[...]
        h0s = nf * no
        h0sA_ref[...] = h0s[:, 0:64]
        h0sB_ref[...] = h0s[:, 64:128]
        x_ref[...] = jax.nn.relu(
            jnp.dot(nf, wx_ref[...], preferred_element_type=_f32) + bx_ref[...]
        )
        nrm_ref[...] = jnp.concatenate([no, ni], axis=1)

    return _tc_call(
        f,
        (
            jax.ShapeDtypeStruct((_NP, 64), _f32),
            jax.ShapeDtypeStruct((_NP, 64), _f32),
            jax.ShapeDtypeStruct((_NP, 64), _f32),
            jax.ShapeDtypeStruct((_NP, 2), _f32),
        ),
        degp, nfp, Wx, bx,
    )


def _tc_layer(aggs, nrm, W, b):
    def f(a_ref, nrm_ref, w_ref, b_ref, outA_ref, outB_ref):
        no = nrm_ref[:, 0:1]
        ni = nrm_ref[:, 1:2]
        p = jnp.concatenate([a_ref[0:_NP, :], a_ref[_NP:2 * _NP, :]], axis=1)
        h = jax.nn.relu(
            jnp.dot(p * ni, w_ref[...], preferred_element_type=_f32)
            + b_ref[...]
        )
        h = h * no
        outA_ref[...] = h[:, 0:64]
        outB_ref[...] = h[:, 64:128]

    return _tc_call(
        f,
        (
            jax.ShapeDtypeStruct((_NP, 64), _f32),
            jax.ShapeDtypeStruct((_NP, 64), _f32),
        ),
        aggs, nrm, W, b,
    )


def _tc_layer2(aggs, nrm, W1, b1, W2):
    def f(a_ref, nrm_ref, w1_ref, b1_ref, w2_ref, outA_ref, outB_ref):
        no = nrm_ref[:, 0:1]
        ni = nrm_ref[:, 1:2]
        p = jnp.concatenate([a_ref[0:_NP, :], a_ref[_NP:2 * _NP, :]], axis=1)
        h = jax.nn.relu(
            jnp.dot(p * ni, w1_ref[...], preferred_element_type=_f32)
            + b1_ref[...]
        )
        t2 = jnp.dot(h, w2_ref[...], preferred_element_type=_f32) * no
        outA_ref[...] = t2[:, 0:16]
        outB_ref[...] = t2[:, 16:32]

    return _tc_call(
        f,
        (
            jax.ShapeDtypeStruct((_NP, 16), _f32),
            jax.ShapeDtypeStruct((_NP, 16), _f32),
        ),
        aggs, nrm, W1, b1, W2,
    )


def _tc_aspect(aggs, nrm, x, b2, W3, We, be):
    def f(a_ref, nrm_ref, x_ref, b2_ref, w3_ref, we_ref, be_ref,
          asp_ref, t3_ref, sb_ref):
        no = nrm_ref[:, 0:1]
        ni = nrm_ref[:, 1:2]
        agg = jnp.concatenate([a_ref[0:_NP, :], a_ref[_NP:2 * _NP, :]], axis=1)
        ae = agg * ni + b2_ref[...]
        m = jnp.max(ae, axis=1, keepdims=True)
        lse = m + jnp.log(jnp.sum(jnp.exp(ae - m), axis=1, keepdims=True))
        asp = ae - lse
        asp_ref[...] = asp
        t3_ref[...] = jnp.dot(ae, w3_ref[...], preferred_element_type=_f32) * no
        we = we_ref[...]
        x = x_ref[...]
        ss = (
            jnp.dot(x, we[0:64, :], preferred_element_type=_f32)
            + jnp.dot(asp, we[128:160, :], preferred_element_type=_f32)
            + be_ref[...]
        )
        sd = (
            jnp.dot(x, we[64:128, :], preferred_element_type=_f32)
            + jnp.dot(asp, we[160:192, :], preferred_element_type=_f32)
        )
        sb_ref[...] = jnp.concatenate([ss, sd], axis=1)

    return _tc_call(
        f,
        (
            jax.ShapeDtypeStruct((_NP, 32), _f32),
            jax.ShapeDtypeStruct((_NP, 16), _f32),
            jax.ShapeDtypeStruct((_NP, 2), _f32),
        ),
        aggs, nrm, x, b2, W3, We, be,
    )


def _tc_logits(aggs, nrm, b3):
    def f(a_ref, nrm_ref, b3_ref, out_ref):
        agg = a_ref[0:_NP, :] + a_ref[_NP:2 * _NP, :]
        out_ref[...] = agg * nrm_ref[:, 1:2] + b3_ref[...]

    return _tc_call(f, jax.ShapeDtypeStruct((_NP, 16), _f32), aggs, nrm, b3)


def kernel(node_features, edge_index, edge_index_neg,
           W0, b0, W1, b1, W2, b2, W3, b3, Wx, bx, We, be):
    i32 = jnp.int32
    pad = jnp.full((_EPAD - _E,), _N, i32)
    src = jnp.concatenate([edge_index[0], pad])
    dst = jnp.concatenate([edge_index[1], pad])
    src3 = src.reshape(32, _TCH, _C)
    dst3 = dst.reshape(32, _TCH, _C)
    src3k = src.reshape(32, 20, 512)
    dst3k = dst.reshape(32, 80, _C)
    srcL2 = src.reshape(16, 80, 256)
    dstL = dst.reshape(16, 160, _C)
    srcL4 = src.reshape(16, 40, 512)
    nsrc = jnp.concatenate([edge_index_neg[0], pad])
    ndst = jnp.concatenate([edge_index_neg[1], pad])
    esrc = jnp.concatenate([src, nsrc])
    edst = jnp.concatenate([dst, ndst])

    nfp = jnp.zeros((_NP, 128), _f32).at[:_N].set(node_features)

    degp = _sc_degrees(src3, dst3).reshape(4, _NP).T
    h0sA, h0sB, x, nrm = _tc_pre(degp, nfp, Wx, bx)

    agg0 = _sc_message_pass_split(h0sA, h0sB, srcL2, dstL, 64, 2, 80)
    h1sA, h1sB = _tc_layer(agg0, nrm, W0, b0)
    agg1 = _sc_message_pass_split(h1sA, h1sB, srcL2, dstL, 64, 2, 80)
    t2A, t2B = _tc_layer2(agg1, nrm, W1, b1, W2)
    agg2 = _sc_message_pass_split(t2A, t2B, srcL4, dstL, 16, 4, 40)
    asp, t3, sboth = _tc_aspect(agg2, nrm, x, b2, W3, We, be)
    agg3 = _sc_message_pass_edges(t3, src3k, dst3k, 16, 4, 20)
    logits = _tc_logits(agg3, nrm, b3)

    preds = _sc_edge_scores(sboth[:, 0].reshape(_NP), sboth[:, 1].reshape(_NP),
                            esrc, edst)
    e_pred_pos = preds[:_E, None]
    e_pred_neg = preds[_EPAD:_EPAD + _E, None]
    return (e_pred_pos, e_pred_neg, asp[:_N], logits[:_N])
